# Initial kernel scaffold; baseline (speedup 1.0000x reference)
import jax
import jax.numpy as jnp
from jax.experimental import pallas as pl


def kernel(query, keys):
    raise NotImplementedError("write your pallas kernel here")


# speedup vs baseline: 12.8046x; 12.8046x over previous
import functools

import jax
import jax.numpy as jnp
from jax.experimental import pallas as pl
from jax.experimental.pallas import tpu as pltpu


def _pass1(qt_ref, keys_ref, cat_ref, sm_ref, colmax_ref, colsum_ref,
           comp_ref, sep_ref, *, n_total):
    i = pl.program_id(0)
    nt = pl.num_programs(0)
    q = qt_ref[...]
    keys = keys_ref[...]
    t, d = q.shape
    m = keys.shape[0]

    nrm = jnp.sqrt(jnp.sum(q * q, axis=1, keepdims=True))
    qn = q / jnp.maximum(nrm, 1e-12)

    score = jax.lax.dot_general(qn, keys, (((1,), (1,)), ((), ())),
                                preferred_element_type=jnp.float32)

    rmax = jnp.max(score, axis=1, keepdims=True)
    er = jnp.exp(score - rmax)
    rsum = jnp.sum(er, axis=1, keepdims=True)
    smv = er / rsum
    sm_ref[...] = smv
    cm = jnp.dot(smv, keys, preferred_element_type=jnp.float32)
    cat_ref[:, :d] = qn
    cat_ref[:, d:] = cm

    iota = jax.lax.broadcasted_iota(jnp.int32, (t, m), 1)
    i1 = jnp.min(jnp.where(score == rmax, iota, m), axis=1, keepdims=True)
    oh1 = (iota == i1).astype(jnp.float32)
    pos = jnp.dot(oh1, keys, preferred_element_type=jnp.float32)
    masked = jnp.where(iota == i1, -jnp.inf, score)
    m2 = jnp.max(masked, axis=1, keepdims=True)
    i2 = jnp.min(jnp.where(masked == m2, iota, m), axis=1, keepdims=True)
    oh2 = (iota == i2).astype(jnp.float32)
    neg = jnp.dot(oh2, keys, preferred_element_type=jnp.float32)

    dpos = qn - pos
    comp_part = jnp.sum(dpos * dpos)
    dp = jnp.sqrt(jnp.sum((dpos + 1e-6) ** 2, axis=1))
    dn = jnp.sqrt(jnp.sum((qn - neg + 1e-6) ** 2, axis=1))
    sep_part = jnp.sum(jnp.maximum(dp - dn + 1.0, 0.0))

    cmax_t = jnp.max(score, axis=0, keepdims=True)

    @pl.when(i == 0)
    def _():
        colmax_ref[...] = jnp.full((1, m), -jnp.inf, jnp.float32)
        colsum_ref[...] = jnp.zeros((1, m), jnp.float32)
        comp_ref[0, 0] = 0.0
        sep_ref[0, 0] = 0.0

    old = colmax_ref[...]
    new = jnp.maximum(old, cmax_t)
    colmax_ref[...] = new
    colsum_ref[...] = (colsum_ref[...] * jnp.exp(old - new)
                       + jnp.sum(jnp.exp(score - new), axis=0, keepdims=True))
    comp_ref[0, 0] = comp_ref[0, 0] + comp_part
    sep_ref[0, 0] = sep_ref[0, 0] + sep_part

    @pl.when(i == nt - 1)
    def _():
        comp_ref[0, 0] = comp_ref[0, 0] / float(n_total * d)
        sep_ref[0, 0] = sep_ref[0, 0] / float(n_total)


def _pass2(cat_ref, keys_ref, colmax_ref, colsum_ref, sq_ref, um_ref):
    i = pl.program_id(0)
    nt = pl.num_programs(0)
    keys = keys_ref[...]
    m, d = keys.shape
    qn = cat_ref[:, :d]
    t = qn.shape[0]

    score = jax.lax.dot_general(qn, keys, (((1,), (1,)), ((), ())),
                                preferred_element_type=jnp.float32)
    cmax = colmax_ref[...]
    csum = colsum_ref[...]
    e = jnp.exp(score - cmax)
    sq_ref[...] = e / csum

    rmax = jnp.max(score, axis=1, keepdims=True)
    iota = jax.lax.broadcasted_iota(jnp.int32, (t, m), 1)
    i1 = jnp.min(jnp.where(score == rmax, iota, m), axis=1, keepdims=True)
    oh = (iota == i1).astype(jnp.float32)
    wgt = jnp.sum(oh * e, axis=1, keepdims=True)
    wq = wgt * qn
    part = jax.lax.dot_general(oh, wq, (((0,), (0,)), ((), ())),
                               preferred_element_type=jnp.float32)

    @pl.when(i == 0)
    def _():
        um_ref[...] = jnp.zeros((m, d), jnp.float32)

    um_ref[...] = um_ref[...] + part

    @pl.when(i == nt - 1)
    def _():
        um = um_ref[...] + keys
        nrm = jnp.sqrt(jnp.sum(um * um, axis=1, keepdims=True))
        um_ref[...] = um / jnp.maximum(nrm, 1e-12)


def kernel(query, keys):
    b, d, h, w = query.shape
    m = keys.shape[0]
    n = b * h * w
    qt = jnp.transpose(query, (0, 2, 3, 1)).reshape(n, d)

    tile = 256
    nt = n // tile

    cat, sm, colmax, colsum, comp, sep = pl.pallas_call(
        functools.partial(_pass1, n_total=n),
        grid=(nt,),
        in_specs=[
            pl.BlockSpec((tile, d), lambda i: (i, 0)),
            pl.BlockSpec((m, d), lambda i: (0, 0)),
        ],
        out_specs=[
            pl.BlockSpec((tile, 2 * d), lambda i: (i, 0)),
            pl.BlockSpec((tile, m), lambda i: (i, 0)),
            pl.BlockSpec((1, m), lambda i: (0, 0)),
            pl.BlockSpec((1, m), lambda i: (0, 0)),
            pl.BlockSpec(memory_space=pltpu.SMEM),
            pl.BlockSpec(memory_space=pltpu.SMEM),
        ],
        out_shape=[
            jax.ShapeDtypeStruct((n, 2 * d), jnp.float32),
            jax.ShapeDtypeStruct((n, m), jnp.float32),
            jax.ShapeDtypeStruct((1, m), jnp.float32),
            jax.ShapeDtypeStruct((1, m), jnp.float32),
            jax.ShapeDtypeStruct((1, 1), jnp.float32),
            jax.ShapeDtypeStruct((1, 1), jnp.float32),
        ],
    )(qt, keys)

    sq, um = pl.pallas_call(
        _pass2,
        grid=(nt,),
        in_specs=[
            pl.BlockSpec((tile, 2 * d), lambda i: (i, 0)),
            pl.BlockSpec((m, d), lambda i: (0, 0)),
            pl.BlockSpec((1, m), lambda i: (0, 0)),
            pl.BlockSpec((1, m), lambda i: (0, 0)),
        ],
        out_specs=[
            pl.BlockSpec((tile, m), lambda i: (i, 0)),
            pl.BlockSpec((m, d), lambda i: (0, 0)),
        ],
        out_shape=[
            jax.ShapeDtypeStruct((n, m), jnp.float32),
            jax.ShapeDtypeStruct((m, d), jnp.float32),
        ],
    )(cat, keys, colmax, colsum)

    uq = jnp.transpose(cat.reshape(b, h, w, 2 * d), (0, 3, 1, 2))
    uo = jnp.transpose(cat[:, d:].reshape(b, h, w, d), (0, 3, 1, 2))
    return (uq, uo, um, sq, sm, sep.reshape(()), comp.reshape(()))


# speedup vs baseline: 16.2102x; 1.2660x over previous
import functools

import jax
import jax.numpy as jnp
from jax.experimental import pallas as pl
from jax.experimental.pallas import tpu as pltpu


def _pass1(qt_ref, kt_ref, cat_ref, sm_ref, colmax_ref, colsum_ref,
           comp_ref, sep_ref, *, n_total):
    i = pl.program_id(0)
    nt = pl.num_programs(0)
    q = qt_ref[...]
    kt = kt_ref[...]
    t, d = q.shape
    m = kt.shape[1]

    qs2 = jnp.sum(q * q, axis=1, keepdims=True)
    qn = q / jnp.maximum(jnp.sqrt(qs2), 1e-12)
    qnn = jnp.sum(qn * qn, axis=1, keepdims=True)
    qs = jnp.sum(qn, axis=1, keepdims=True)

    kn2 = jnp.sum(kt * kt, axis=0, keepdims=True)
    ksum = jnp.sum(kt, axis=0, keepdims=True)
    combo = kn2 - 2e-6 * ksum

    score = jnp.dot(qn, kt, preferred_element_type=jnp.float32)

    rmax = jnp.max(score, axis=1, keepdims=True)
    er = jnp.exp(score - rmax)
    rsum = jnp.sum(er, axis=1, keepdims=True)
    smv = er * (1.0 / rsum)
    sm_ref[...] = smv
    cm = jax.lax.dot_general(smv, kt, (((1,), (1,)), ((), ())),
                             preferred_element_type=jnp.float32)
    cat_ref[:, :d] = qn
    cat_ref[:, d:] = cm

    oh1 = score == rmax
    kn2g = jnp.sum(jnp.where(oh1, kn2, 0.0), axis=1, keepdims=True)
    ksumg = jnp.sum(jnp.where(oh1, ksum, 0.0), axis=1, keepdims=True)
    masked = jnp.where(oh1, -jnp.inf, score)
    m2 = jnp.max(masked, axis=1, keepdims=True)
    oh2 = masked == m2
    cg2 = jnp.sum(jnp.where(oh2, combo, 0.0), axis=1, keepdims=True)

    base = qnn + 2e-6 * qs + 6.4e-11
    comp_part = jnp.sum(qnn - 2.0 * rmax + kn2g)
    dp = jnp.sqrt(jnp.maximum(base - 2.0 * rmax + kn2g - 2e-6 * ksumg, 0.0))
    dn = jnp.sqrt(jnp.maximum(base - 2.0 * m2 + cg2, 0.0))
    sep_part = jnp.sum(jnp.maximum(dp - dn + 1.0, 0.0))

    eS = er * jnp.exp(rmax)
    ctile_max = jnp.max(score, axis=0, keepdims=True)
    ctile_sum = jnp.sum(eS, axis=0, keepdims=True)

    @pl.when(i == 0)
    def _():
        colmax_ref[...] = jnp.full((1, m), -jnp.inf, jnp.float32)
        colsum_ref[...] = jnp.zeros((1, m), jnp.float32)
        comp_ref[0, 0] = 0.0
        sep_ref[0, 0] = 0.0

    colmax_ref[...] = jnp.maximum(colmax_ref[...], ctile_max)
    colsum_ref[...] = colsum_ref[...] + ctile_sum
    comp_ref[0, 0] = comp_ref[0, 0] + comp_part
    sep_ref[0, 0] = sep_ref[0, 0] + sep_part

    @pl.when(i == nt - 1)
    def _():
        comp_ref[0, 0] = comp_ref[0, 0] / float(n_total * d)
        sep_ref[0, 0] = sep_ref[0, 0] / float(n_total)


def _pass2(cat_ref, kt_ref, keys_ref, colmax_ref, colsum_ref, sq_ref, um_ref):
    i = pl.program_id(0)
    nt = pl.num_programs(0)
    kt = kt_ref[...]
    d, m = kt.shape
    qn = cat_ref[:, :d]

    score = jnp.dot(qn, kt, preferred_element_type=jnp.float32)
    e = jnp.exp(score)
    sq_ref[...] = e * (1.0 / colsum_ref[...])

    rmax = jnp.max(score, axis=1, keepdims=True)
    oh1 = score == rmax
    cmaxg = jnp.sum(jnp.where(oh1, colmax_ref[...], 0.0),
                    axis=1, keepdims=True)
    wgt = jnp.exp(rmax - cmaxg)
    ohf = jnp.where(oh1, 1.0, 0.0)
    wq = wgt * qn
    part = jax.lax.dot_general(ohf, wq, (((0,), (0,)), ((), ())),
                               preferred_element_type=jnp.float32)

    @pl.when(i == 0)
    def _():
        um_ref[...] = jnp.zeros((m, d), jnp.float32)

    um_ref[...] = um_ref[...] + part

    @pl.when(i == nt - 1)
    def _():
        um = um_ref[...] + keys_ref[...]
        nrm = jnp.sqrt(jnp.sum(um * um, axis=1, keepdims=True))
        um_ref[...] = um / jnp.maximum(nrm, 1e-12)


def kernel(query, keys):
    b, d, h, w = query.shape
    m = keys.shape[0]
    n = b * h * w
    qt = jnp.transpose(query, (0, 2, 3, 1)).reshape(n, d)
    kt = keys.T

    tile = 256
    nt = n // tile

    cat, sm, colmax, colsum, comp, sep = pl.pallas_call(
        functools.partial(_pass1, n_total=n),
        grid=(nt,),
        in_specs=[
            pl.BlockSpec((tile, d), lambda i: (i, 0)),
            pl.BlockSpec((d, m), lambda i: (0, 0)),
        ],
        out_specs=[
            pl.BlockSpec((tile, 2 * d), lambda i: (i, 0)),
            pl.BlockSpec((tile, m), lambda i: (i, 0)),
            pl.BlockSpec((1, m), lambda i: (0, 0)),
            pl.BlockSpec((1, m), lambda i: (0, 0)),
            pl.BlockSpec(memory_space=pltpu.SMEM),
            pl.BlockSpec(memory_space=pltpu.SMEM),
        ],
        out_shape=[
            jax.ShapeDtypeStruct((n, 2 * d), jnp.float32),
            jax.ShapeDtypeStruct((n, m), jnp.float32),
            jax.ShapeDtypeStruct((1, m), jnp.float32),
            jax.ShapeDtypeStruct((1, m), jnp.float32),
            jax.ShapeDtypeStruct((1, 1), jnp.float32),
            jax.ShapeDtypeStruct((1, 1), jnp.float32),
        ],
    )(qt, kt)

    sq, um = pl.pallas_call(
        _pass2,
        grid=(nt,),
        in_specs=[
            pl.BlockSpec((tile, 2 * d), lambda i: (i, 0)),
            pl.BlockSpec((d, m), lambda i: (0, 0)),
            pl.BlockSpec((m, d), lambda i: (0, 0)),
            pl.BlockSpec((1, m), lambda i: (0, 0)),
            pl.BlockSpec((1, m), lambda i: (0, 0)),
        ],
        out_specs=[
            pl.BlockSpec((tile, m), lambda i: (i, 0)),
            pl.BlockSpec((m, d), lambda i: (0, 0)),
        ],
        out_shape=[
            jax.ShapeDtypeStruct((n, m), jnp.float32),
            jax.ShapeDtypeStruct((m, d), jnp.float32),
        ],
    )(cat, kt, keys, colmax, colsum)

    uq = jnp.transpose(cat.reshape(b, h, w, 2 * d), (0, 3, 1, 2))
    uo = jnp.transpose(cat[:, d:].reshape(b, h, w, d), (0, 3, 1, 2))
    return (uq, uo, um, sq, sm, sep.reshape(()), comp.reshape(()))


# speedup vs baseline: 20.3556x; 1.2557x over previous
import functools

import jax
import jax.numpy as jnp
from jax.experimental import pallas as pl
from jax.experimental.pallas import tpu as pltpu


def _pass1(qt_ref, kt_ref, cat_ref, sm_ref, colmax_ref, colsum_ref,
           comp_ref, sep_ref, *, n_total):
    i = pl.program_id(0)
    nt = pl.num_programs(0)
    q = qt_ref[...]
    kt = kt_ref[...]
    t, d = q.shape
    m = kt.shape[1]

    qs2 = jnp.sum(q * q, axis=1, keepdims=True)
    qn = q / jnp.maximum(jnp.sqrt(qs2), 1e-12)
    qnn = jnp.sum(qn * qn, axis=1, keepdims=True)
    qs = jnp.sum(qn, axis=1, keepdims=True)

    kn2 = jnp.sum(kt * kt, axis=0, keepdims=True)
    ksum = jnp.sum(kt, axis=0, keepdims=True)
    combo = kn2 - 2e-6 * ksum

    score = jnp.dot(qn, kt, preferred_element_type=jnp.float32)

    rmax = jnp.max(score, axis=1, keepdims=True)
    er = jnp.exp(score - rmax)
    rsum = jnp.sum(er, axis=1, keepdims=True)
    smv = er * (1.0 / rsum)
    sm_ref[...] = smv
    cm = jax.lax.dot_general(smv, kt, (((1,), (1,)), ((), ())),
                             preferred_element_type=jnp.float32)
    cat_ref[:, :d] = qn
    cat_ref[:, d:] = cm

    oh1 = score == rmax
    kn2g = jnp.sum(jnp.where(oh1, kn2, 0.0), axis=1, keepdims=True)
    ksumg = jnp.sum(jnp.where(oh1, ksum, 0.0), axis=1, keepdims=True)
    masked = jnp.where(oh1, -jnp.inf, score)
    m2 = jnp.max(masked, axis=1, keepdims=True)
    oh2 = masked == m2
    cg2 = jnp.sum(jnp.where(oh2, combo, 0.0), axis=1, keepdims=True)

    base = qnn + 2e-6 * qs + 6.4e-11
    comp_part = jnp.sum(qnn - 2.0 * rmax + kn2g)
    dp = jnp.sqrt(jnp.maximum(base - 2.0 * rmax + kn2g - 2e-6 * ksumg, 0.0))
    dn = jnp.sqrt(jnp.maximum(base - 2.0 * m2 + cg2, 0.0))
    sep_part = jnp.sum(jnp.maximum(dp - dn + 1.0, 0.0))

    eS = er * jnp.exp(rmax)
    ctile_max = jnp.max(score, axis=0, keepdims=True)
    ctile_sum = jnp.sum(eS, axis=0, keepdims=True)

    @pl.when(i == 0)
    def _():
        colmax_ref[...] = jnp.full((1, m), -jnp.inf, jnp.float32)
        colsum_ref[...] = jnp.zeros((1, m), jnp.float32)
        comp_ref[0, 0] = 0.0
        sep_ref[0, 0] = 0.0

    colmax_ref[...] = jnp.maximum(colmax_ref[...], ctile_max)
    colsum_ref[...] = colsum_ref[...] + ctile_sum
    comp_ref[0, 0] = comp_ref[0, 0] + comp_part
    sep_ref[0, 0] = sep_ref[0, 0] + sep_part

    @pl.when(i == nt - 1)
    def _():
        comp_ref[0, 0] = comp_ref[0, 0] / float(n_total * d)
        sep_ref[0, 0] = sep_ref[0, 0] / float(n_total)


def _pass2(cat_ref, kt_ref, keys_ref, colmax_ref, colsum_ref, sq_ref, um_ref):
    i = pl.program_id(0)
    nt = pl.num_programs(0)
    kt = kt_ref[...]
    d, m = kt.shape
    qn = cat_ref[:, :d]

    score = jnp.dot(qn, kt, preferred_element_type=jnp.float32)
    e = jnp.exp(score)
    sq_ref[...] = e * (1.0 / colsum_ref[...])

    rmax = jnp.max(score, axis=1, keepdims=True)
    oh1 = score == rmax
    cmaxg = jnp.sum(jnp.where(oh1, colmax_ref[...], 0.0),
                    axis=1, keepdims=True)
    wgt = jnp.exp(rmax - cmaxg)
    ohf = jnp.where(oh1, 1.0, 0.0)
    wq = wgt * qn
    part = jax.lax.dot_general(ohf, wq, (((0,), (0,)), ((), ())),
                               preferred_element_type=jnp.float32)

    @pl.when(i == 0)
    def _():
        um_ref[...] = jnp.zeros((m, d), jnp.float32)

    um_ref[...] = um_ref[...] + part

    @pl.when(i == nt - 1)
    def _():
        um = um_ref[...] + keys_ref[...]
        nrm = jnp.sqrt(jnp.sum(um * um, axis=1, keepdims=True))
        um_ref[...] = um / jnp.maximum(nrm, 1e-12)


def kernel(query, keys):
    b, d, h, w = query.shape
    m = keys.shape[0]
    n = b * h * w
    qt = jnp.transpose(query, (0, 2, 3, 1)).reshape(n, d)
    kt = keys.T

    tile = 512
    nt = n // tile

    cat, sm, colmax, colsum, comp, sep = pl.pallas_call(
        functools.partial(_pass1, n_total=n),
        grid=(nt,),
        in_specs=[
            pl.BlockSpec((tile, d), lambda i: (i, 0)),
            pl.BlockSpec((d, m), lambda i: (0, 0)),
        ],
        out_specs=[
            pl.BlockSpec((tile, 2 * d), lambda i: (i, 0)),
            pl.BlockSpec((tile, m), lambda i: (i, 0)),
            pl.BlockSpec((1, m), lambda i: (0, 0)),
            pl.BlockSpec((1, m), lambda i: (0, 0)),
            pl.BlockSpec(memory_space=pltpu.SMEM),
            pl.BlockSpec(memory_space=pltpu.SMEM),
        ],
        out_shape=[
            jax.ShapeDtypeStruct((n, 2 * d), jnp.float32),
            jax.ShapeDtypeStruct((n, m), jnp.float32),
            jax.ShapeDtypeStruct((1, m), jnp.float32),
            jax.ShapeDtypeStruct((1, m), jnp.float32),
            jax.ShapeDtypeStruct((1, 1), jnp.float32),
            jax.ShapeDtypeStruct((1, 1), jnp.float32),
        ],
    )(qt, kt)

    sq, um = pl.pallas_call(
        _pass2,
        grid=(nt,),
        in_specs=[
            pl.BlockSpec((tile, 2 * d), lambda i: (i, 0)),
            pl.BlockSpec((d, m), lambda i: (0, 0)),
            pl.BlockSpec((m, d), lambda i: (0, 0)),
            pl.BlockSpec((1, m), lambda i: (0, 0)),
            pl.BlockSpec((1, m), lambda i: (0, 0)),
        ],
        out_specs=[
            pl.BlockSpec((tile, m), lambda i: (i, 0)),
            pl.BlockSpec((m, d), lambda i: (0, 0)),
        ],
        out_shape=[
            jax.ShapeDtypeStruct((n, m), jnp.float32),
            jax.ShapeDtypeStruct((m, d), jnp.float32),
        ],
    )(cat, kt, keys, colmax, colsum)

    uq = jnp.transpose(cat.reshape(b, h, w, 2 * d), (0, 3, 1, 2))
    uo = jnp.transpose(cat[:, d:].reshape(b, h, w, d), (0, 3, 1, 2))
    return (uq, uo, um, sq, sm, sep.reshape(()), comp.reshape(()))


# speedup vs baseline: 22.6430x; 1.1124x over previous
import functools

import jax
import jax.numpy as jnp
from jax.experimental import pallas as pl
from jax.experimental.pallas import tpu as pltpu


def _pass1(qt_ref, kt_ref, cat_ref, sm_ref, colmax_ref, colsum_ref,
           comp_ref, sep_ref, *, n_total):
    i = pl.program_id(0)
    nt = pl.num_programs(0)
    q = qt_ref[...]
    kt = kt_ref[...]
    t, d = q.shape
    m = kt.shape[1]

    qs2 = jnp.sum(q * q, axis=1, keepdims=True)
    qn = q / jnp.maximum(jnp.sqrt(qs2), 1e-12)
    qnn = jnp.sum(qn * qn, axis=1, keepdims=True)
    qs = jnp.sum(qn, axis=1, keepdims=True)

    kn2 = jnp.sum(kt * kt, axis=0, keepdims=True)
    ksum = jnp.sum(kt, axis=0, keepdims=True)
    combo = kn2 - 2e-6 * ksum

    score = jnp.dot(qn, kt, preferred_element_type=jnp.float32)

    rmax = jnp.max(score, axis=1, keepdims=True)
    er = jnp.exp(score - rmax)
    rsum = jnp.sum(er, axis=1, keepdims=True)
    smv = er * (1.0 / rsum)
    sm_ref[...] = smv
    cm = jax.lax.dot_general(smv, kt, (((1,), (1,)), ((), ())),
                             preferred_element_type=jnp.float32)
    cat_ref[:, :d] = qn
    cat_ref[:, d:] = cm

    oh1 = score == rmax
    kn2g = jnp.sum(jnp.where(oh1, kn2, 0.0), axis=1, keepdims=True)
    ksumg = jnp.sum(jnp.where(oh1, ksum, 0.0), axis=1, keepdims=True)
    masked = jnp.where(oh1, -jnp.inf, score)
    m2 = jnp.max(masked, axis=1, keepdims=True)
    oh2 = masked == m2
    cg2 = jnp.sum(jnp.where(oh2, combo, 0.0), axis=1, keepdims=True)

    base = qnn + 2e-6 * qs + 6.4e-11
    comp_part = jnp.sum(qnn - 2.0 * rmax + kn2g)
    dp = jnp.sqrt(jnp.maximum(base - 2.0 * rmax + kn2g - 2e-6 * ksumg, 0.0))
    dn = jnp.sqrt(jnp.maximum(base - 2.0 * m2 + cg2, 0.0))
    sep_part = jnp.sum(jnp.maximum(dp - dn + 1.0, 0.0))

    eS = er * jnp.exp(rmax)
    ctile_max = jnp.max(score, axis=0, keepdims=True)
    ctile_sum = jnp.sum(eS, axis=0, keepdims=True)

    @pl.when(i == 0)
    def _():
        colmax_ref[...] = jnp.full((1, m), -jnp.inf, jnp.float32)
        colsum_ref[...] = jnp.zeros((1, m), jnp.float32)
        comp_ref[0, 0] = 0.0
        sep_ref[0, 0] = 0.0

    colmax_ref[...] = jnp.maximum(colmax_ref[...], ctile_max)
    colsum_ref[...] = colsum_ref[...] + ctile_sum
    comp_ref[0, 0] = comp_ref[0, 0] + comp_part
    sep_ref[0, 0] = sep_ref[0, 0] + sep_part

    @pl.when(i == nt - 1)
    def _():
        comp_ref[0, 0] = comp_ref[0, 0] / float(n_total * d)
        sep_ref[0, 0] = sep_ref[0, 0] / float(n_total)


def _pass2(cat_ref, kt_ref, keys_ref, colmax_ref, colsum_ref, sq_ref, um_ref):
    i = pl.program_id(0)
    nt = pl.num_programs(0)
    kt = kt_ref[...]
    d, m = kt.shape
    qn = cat_ref[:, :d]

    score = jnp.dot(qn, kt, preferred_element_type=jnp.float32)
    e = jnp.exp(score)
    sq_ref[...] = e * (1.0 / colsum_ref[...])

    rmax = jnp.max(score, axis=1, keepdims=True)
    oh1 = score == rmax
    cmaxg = jnp.sum(jnp.where(oh1, colmax_ref[...], 0.0),
                    axis=1, keepdims=True)
    wgt = jnp.exp(rmax - cmaxg)
    ohf = jnp.where(oh1, 1.0, 0.0)
    wq = wgt * qn
    part = jax.lax.dot_general(ohf, wq, (((0,), (0,)), ((), ())),
                               preferred_element_type=jnp.float32)

    @pl.when(i == 0)
    def _():
        um_ref[...] = jnp.zeros((m, d), jnp.float32)

    um_ref[...] = um_ref[...] + part

    @pl.when(i == nt - 1)
    def _():
        um = um_ref[...] + keys_ref[...]
        nrm = jnp.sqrt(jnp.sum(um * um, axis=1, keepdims=True))
        um_ref[...] = um / jnp.maximum(nrm, 1e-12)


def kernel(query, keys):
    b, d, h, w = query.shape
    m = keys.shape[0]
    n = b * h * w
    qt = jnp.transpose(query, (0, 2, 3, 1)).reshape(n, d)
    kt = keys.T

    tile = 1024
    nt = n // tile

    cat, sm, colmax, colsum, comp, sep = pl.pallas_call(
        functools.partial(_pass1, n_total=n),
        grid=(nt,),
        in_specs=[
            pl.BlockSpec((tile, d), lambda i: (i, 0)),
            pl.BlockSpec((d, m), lambda i: (0, 0)),
        ],
        out_specs=[
            pl.BlockSpec((tile, 2 * d), lambda i: (i, 0)),
            pl.BlockSpec((tile, m), lambda i: (i, 0)),
            pl.BlockSpec((1, m), lambda i: (0, 0)),
            pl.BlockSpec((1, m), lambda i: (0, 0)),
            pl.BlockSpec(memory_space=pltpu.SMEM),
            pl.BlockSpec(memory_space=pltpu.SMEM),
        ],
        out_shape=[
            jax.ShapeDtypeStruct((n, 2 * d), jnp.float32),
            jax.ShapeDtypeStruct((n, m), jnp.float32),
            jax.ShapeDtypeStruct((1, m), jnp.float32),
            jax.ShapeDtypeStruct((1, m), jnp.float32),
            jax.ShapeDtypeStruct((1, 1), jnp.float32),
            jax.ShapeDtypeStruct((1, 1), jnp.float32),
        ],
    )(qt, kt)

    sq, um = pl.pallas_call(
        _pass2,
        grid=(nt,),
        in_specs=[
            pl.BlockSpec((tile, 2 * d), lambda i: (i, 0)),
            pl.BlockSpec((d, m), lambda i: (0, 0)),
            pl.BlockSpec((m, d), lambda i: (0, 0)),
            pl.BlockSpec((1, m), lambda i: (0, 0)),
            pl.BlockSpec((1, m), lambda i: (0, 0)),
        ],
        out_specs=[
            pl.BlockSpec((tile, m), lambda i: (i, 0)),
            pl.BlockSpec((m, d), lambda i: (0, 0)),
        ],
        out_shape=[
            jax.ShapeDtypeStruct((n, m), jnp.float32),
            jax.ShapeDtypeStruct((m, d), jnp.float32),
        ],
    )(cat, kt, keys, colmax, colsum)

    uq = jnp.transpose(cat.reshape(b, h, w, 2 * d), (0, 3, 1, 2))
    uo = jnp.transpose(cat[:, d:].reshape(b, h, w, d), (0, 3, 1, 2))
    return (uq, uo, um, sq, sm, sep.reshape(()), comp.reshape(()))


# speedup vs baseline: 23.0732x; 1.0190x over previous
import functools

import jax
import jax.numpy as jnp
from jax.experimental import pallas as pl
from jax.experimental.pallas import tpu as pltpu


def _pass1(qt_ref, kt_ref, cat_ref, sm_ref, colmax_ref, colsum_ref,
           comp_ref, sep_ref, *, n_total):
    i = pl.program_id(0)
    nt = pl.num_programs(0)
    q = qt_ref[...]
    kt = kt_ref[...]
    t, d = q.shape
    m = kt.shape[1]

    qs2 = jnp.sum(q * q, axis=1, keepdims=True)
    qn = q / jnp.maximum(jnp.sqrt(qs2), 1e-12)
    qnn = jnp.sum(qn * qn, axis=1, keepdims=True)
    qs = jnp.sum(qn, axis=1, keepdims=True)

    kn2 = jnp.sum(kt * kt, axis=0, keepdims=True)
    ksum = jnp.sum(kt, axis=0, keepdims=True)
    combo = kn2 - 2e-6 * ksum

    score = jnp.dot(qn, kt, preferred_element_type=jnp.float32)

    rmax = jnp.max(score, axis=1, keepdims=True)
    er = jnp.exp(score - rmax)
    rsum = jnp.sum(er, axis=1, keepdims=True)
    smv = er * (1.0 / rsum)
    sm_ref[...] = smv
    cm = jax.lax.dot_general(smv, kt, (((1,), (1,)), ((), ())),
                             preferred_element_type=jnp.float32)
    cat_ref[:, :d] = qn
    cat_ref[:, d:] = cm

    oh1 = score == rmax
    kn2g = jnp.sum(jnp.where(oh1, kn2, 0.0), axis=1, keepdims=True)
    ksumg = jnp.sum(jnp.where(oh1, ksum, 0.0), axis=1, keepdims=True)
    masked = jnp.where(oh1, -jnp.inf, score)
    m2 = jnp.max(masked, axis=1, keepdims=True)
    oh2 = masked == m2
    cg2 = jnp.sum(jnp.where(oh2, combo, 0.0), axis=1, keepdims=True)

    base = qnn + 2e-6 * qs + 6.4e-11
    comp_part = jnp.sum(qnn - 2.0 * rmax + kn2g)
    dp = jnp.sqrt(jnp.maximum(base - 2.0 * rmax + kn2g - 2e-6 * ksumg, 0.0))
    dn = jnp.sqrt(jnp.maximum(base - 2.0 * m2 + cg2, 0.0))
    sep_part = jnp.sum(jnp.maximum(dp - dn + 1.0, 0.0))

    eS = er * jnp.exp(rmax)
    ctile_max = jnp.max(score, axis=0, keepdims=True)
    ctile_sum = jnp.sum(eS, axis=0, keepdims=True)

    @pl.when(i == 0)
    def _():
        colmax_ref[...] = jnp.full((1, m), -jnp.inf, jnp.float32)
        colsum_ref[...] = jnp.zeros((1, m), jnp.float32)
        comp_ref[0, 0] = 0.0
        sep_ref[0, 0] = 0.0

    colmax_ref[...] = jnp.maximum(colmax_ref[...], ctile_max)
    colsum_ref[...] = colsum_ref[...] + ctile_sum
    comp_ref[0, 0] = comp_ref[0, 0] + comp_part
    sep_ref[0, 0] = sep_ref[0, 0] + sep_part

    @pl.when(i == nt - 1)
    def _():
        comp_ref[0, 0] = comp_ref[0, 0] / float(n_total * d)
        sep_ref[0, 0] = sep_ref[0, 0] / float(n_total)


def _pass2(cat_ref, kt_ref, keys_ref, colmax_ref, colsum_ref, sq_ref, um_ref):
    i = pl.program_id(0)
    nt = pl.num_programs(0)
    kt = kt_ref[...]
    d, m = kt.shape
    qn = cat_ref[:, :d]

    score = jnp.dot(qn, kt, preferred_element_type=jnp.float32)
    e = jnp.exp(score)
    sq_ref[...] = e * (1.0 / colsum_ref[...])

    rmax = jnp.max(score, axis=1, keepdims=True)
    oh1 = score == rmax
    cmaxg = jnp.sum(jnp.where(oh1, colmax_ref[...], 0.0),
                    axis=1, keepdims=True)
    wgt = jnp.exp(rmax - cmaxg)
    ohf = jnp.where(oh1, 1.0, 0.0)
    wq = wgt * qn
    part = jax.lax.dot_general(ohf, wq, (((0,), (0,)), ((), ())),
                               preferred_element_type=jnp.float32)

    @pl.when(i == 0)
    def _():
        um_ref[...] = jnp.zeros((m, d), jnp.float32)

    um_ref[...] = um_ref[...] + part

    @pl.when(i == nt - 1)
    def _():
        um = um_ref[...] + keys_ref[...]
        nrm = jnp.sqrt(jnp.sum(um * um, axis=1, keepdims=True))
        um_ref[...] = um / jnp.maximum(nrm, 1e-12)


def kernel(query, keys):
    b, d, h, w = query.shape
    m = keys.shape[0]
    n = b * h * w
    qt = jnp.transpose(query, (0, 2, 3, 1)).reshape(n, d)
    kt = keys.T

    tile = 2048
    nt = n // tile

    cat, sm, colmax, colsum, comp, sep = pl.pallas_call(
        functools.partial(_pass1, n_total=n),
        grid=(nt,),
        in_specs=[
            pl.BlockSpec((tile, d), lambda i: (i, 0)),
            pl.BlockSpec((d, m), lambda i: (0, 0)),
        ],
        out_specs=[
            pl.BlockSpec((tile, 2 * d), lambda i: (i, 0)),
            pl.BlockSpec((tile, m), lambda i: (i, 0)),
            pl.BlockSpec((1, m), lambda i: (0, 0)),
            pl.BlockSpec((1, m), lambda i: (0, 0)),
            pl.BlockSpec(memory_space=pltpu.SMEM),
            pl.BlockSpec(memory_space=pltpu.SMEM),
        ],
        out_shape=[
            jax.ShapeDtypeStruct((n, 2 * d), jnp.float32),
            jax.ShapeDtypeStruct((n, m), jnp.float32),
            jax.ShapeDtypeStruct((1, m), jnp.float32),
            jax.ShapeDtypeStruct((1, m), jnp.float32),
            jax.ShapeDtypeStruct((1, 1), jnp.float32),
            jax.ShapeDtypeStruct((1, 1), jnp.float32),
        ],
    )(qt, kt)

    sq, um = pl.pallas_call(
        _pass2,
        grid=(nt,),
        in_specs=[
            pl.BlockSpec((tile, 2 * d), lambda i: (i, 0)),
            pl.BlockSpec((d, m), lambda i: (0, 0)),
            pl.BlockSpec((m, d), lambda i: (0, 0)),
            pl.BlockSpec((1, m), lambda i: (0, 0)),
            pl.BlockSpec((1, m), lambda i: (0, 0)),
        ],
        out_specs=[
            pl.BlockSpec((tile, m), lambda i: (i, 0)),
            pl.BlockSpec((m, d), lambda i: (0, 0)),
        ],
        out_shape=[
            jax.ShapeDtypeStruct((n, m), jnp.float32),
            jax.ShapeDtypeStruct((m, d), jnp.float32),
        ],
    )(cat, kt, keys, colmax, colsum)

    uq = jnp.transpose(cat.reshape(b, h, w, 2 * d), (0, 3, 1, 2))
    uo = jnp.transpose(cat[:, d:].reshape(b, h, w, d), (0, 3, 1, 2))
    return (uq, uo, um, sq, sm, sep.reshape(()), comp.reshape(()))
